# Initial kernel scaffold; baseline (speedup 1.0000x reference)
import functools

import jax
import jax.numpy as jnp
from jax import lax
from jax.experimental import pallas as pl
from jax.experimental.pallas import tpu as pltpu
from jax.experimental.pallas import tpu_sc as plsc

NC = 2
NS = 16
LANES = 16

CHUNK = 128


def _sc_partial_agg(x, edge_index, n_pad):
    n, d = x.shape
    e = edge_index.shape[1]
    n_chunks = e // CHUNK
    chunks_per_core = n_chunks // NC
    iters = (chunks_per_core + NS - 1) // NS
    rows_per_sub = n_pad // NS
    dummy = n

    zeros = jnp.zeros((rows_per_sub, d), jnp.float32)

    mesh = plsc.VectorSubcoreMesh(core_axis_name="c", subcore_axis_name="s")

    @functools.partial(
        pl.kernel,
        out_type=jax.ShapeDtypeStruct((NC, n_pad, d), jnp.float32),
        mesh=mesh,
        scratch_types=[
            pltpu.VMEM((CHUNK,), jnp.int32),
            pltpu.VMEM((CHUNK,), jnp.int32),
            pltpu.VMEM((CHUNK, d), jnp.float32),
            pltpu.VMEM_SHARED((n_pad, d), jnp.float32),
        ],
    )
    def sc_kernel(x_hbm, ei_hbm, z_hbm, out_hbm, dst_v, src_v, rows_v, acc):
        c = lax.axis_index("c")
        s = lax.axis_index("s")

        pltpu.sync_copy(z_hbm, acc.at[pl.ds(s * rows_per_sub, rows_per_sub)])
        plsc.subcore_barrier()

        @pl.loop(0, iters)
        def _(k):
            l = s + k * NS

            @pl.when(l < chunks_per_core)
            def _():
                base = (c * chunks_per_core + l) * CHUNK
                pltpu.sync_copy(ei_hbm.at[0, pl.ds(base, CHUNK)], dst_v)
                pltpu.sync_copy(ei_hbm.at[1, pl.ds(base, CHUNK)], src_v)

                @pl.loop(0, CHUNK, step=LANES)
                def _(i):
                    dsl = dst_v[pl.ds(i, LANES)]
                    ssl = src_v[pl.ds(i, LANES)]
                    dst_v[pl.ds(i, LANES)] = jnp.where(dsl != ssl, dsl, dummy)

                pltpu.sync_copy(x_hbm.at[src_v], rows_v)
                pltpu.sync_copy(rows_v, acc.at[dst_v], add=True)

        plsc.subcore_barrier()

        pltpu.sync_copy(
            acc.at[pl.ds(s * rows_per_sub, rows_per_sub)],
            out_hbm.at[c, pl.ds(s * rows_per_sub, rows_per_sub)],
        )

    return sc_kernel(x, edge_index, zeros)


def _tc_combine_body(eps_ref, x_ref, p_ref, o_ref):
    scale = 1.0 + eps_ref[0]
    o_ref[...] = scale * x_ref[...] + p_ref[0] + p_ref[1]


def kernel(x, edge_index, eps):
    n, d = x.shape
    n_pad = 10240
    partial = _sc_partial_agg(x, edge_index, n_pad)

    blk = 1000
    grid = (n // blk,)
    out = pl.pallas_call(
        _tc_combine_body,
        grid=grid,
        in_specs=[
            pl.BlockSpec(memory_space=pltpu.SMEM),
            pl.BlockSpec((blk, d), lambda i: (i, 0)),
            pl.BlockSpec((NC, blk, d), lambda i: (0, i, 0)),
        ],
        out_specs=pl.BlockSpec((blk, d), lambda i: (i, 0)),
        out_shape=jax.ShapeDtypeStruct((n, d), jnp.float32),
    )(eps, x, partial[:, :n, :])
    return out


# speedup vs baseline: 7.3924x; 7.3924x over previous
import functools

import jax
import jax.numpy as jnp
from jax import lax
from jax.experimental import pallas as pl
from jax.experimental.pallas import tpu as pltpu
from jax.experimental.pallas import tpu_sc as plsc

NC = 2
NS = 16
LANES = 16

CHUNK = 128


def _sc_partial_agg(x, edge_index, n_pad):
    n, d = x.shape
    e = edge_index.shape[1]
    n_chunks = e // CHUNK
    chunks_per_core = n_chunks // NC
    iters = (chunks_per_core + NS - 1) // NS
    rows_per_sub = n_pad // NS
    dummy = n

    zeros = jnp.zeros((rows_per_sub, d), jnp.float32)

    mesh = plsc.VectorSubcoreMesh(core_axis_name="c", subcore_axis_name="s")

    @functools.partial(
        pl.kernel,
        out_type=jax.ShapeDtypeStruct((NC, n_pad, d), jnp.float32),
        mesh=mesh,
        scratch_types=[
            pltpu.VMEM((CHUNK,), jnp.int32),
            pltpu.VMEM((CHUNK,), jnp.int32),
            pltpu.VMEM((CHUNK, d), jnp.float32),
            pltpu.VMEM_SHARED((n_pad, d), jnp.float32),
        ],
    )
    def sc_kernel(x_hbm, ei_hbm, z_hbm, out_hbm, dst_v, src_v, rows_v, acc):
        c = lax.axis_index("c")
        s = lax.axis_index("s")

        pltpu.sync_copy(z_hbm, acc.at[pl.ds(s * rows_per_sub, rows_per_sub)])
        plsc.subcore_barrier()

        @pl.loop(0, iters)
        def _(k):
            l = s + k * NS

            @pl.when(l < chunks_per_core)
            def _():
                base = (c * chunks_per_core + l) * CHUNK
                pltpu.sync_copy(ei_hbm.at[0, pl.ds(base, CHUNK)], dst_v)
                pltpu.sync_copy(ei_hbm.at[1, pl.ds(base, CHUNK)], src_v)

                @pl.loop(0, CHUNK, step=LANES)
                def _(i):
                    dsl = dst_v[pl.ds(i, LANES)]
                    ssl = src_v[pl.ds(i, LANES)]
                    dst_v[pl.ds(i, LANES)] = jnp.where(dsl != ssl, dsl, dummy)

                pltpu.sync_copy(x_hbm.at[src_v], rows_v)
                pltpu.sync_copy(rows_v, acc.at[dst_v], add=True)

        plsc.subcore_barrier()

        pltpu.sync_copy(
            acc.at[pl.ds(s * rows_per_sub, rows_per_sub)],
            out_hbm.at[c, pl.ds(s * rows_per_sub, rows_per_sub)],
        )

    return sc_kernel(x, edge_index, zeros)


def _tc_combine_body(eps_ref, x_ref, p_ref, o_ref):
    scale = 1.0 + eps_ref[0]
    o_ref[...] = scale * x_ref[...] + p_ref[0] + p_ref[1]


def kernel(x, edge_index, eps):
    n, d = x.shape
    n_pad = 10240
    partial = _sc_partial_agg(x, edge_index, n_pad)

    blk = 1000
    grid = (n // blk,)
    out = pl.pallas_call(
        _tc_combine_body,
        grid=grid,
        in_specs=[
            pl.BlockSpec(memory_space=pltpu.SMEM),
            pl.BlockSpec((blk, d), lambda i: (i, 0)),
            pl.BlockSpec((NC, blk, d), lambda i: (0, i, 0)),
        ],
        out_specs=pl.BlockSpec((blk, d), lambda i: (i, 0)),
        out_shape=jax.ShapeDtypeStruct((n, d), jnp.float32),
    )(eps, x, partial)
    return out
